# Initial kernel scaffold; baseline (speedup 1.0000x reference)
import jax
import jax.numpy as jnp
from jax.experimental import pallas as pl


def kernel(user_emb, item_emb, W_c1, b_c1, W_c2, b_c2, W_c3, b_c3, W_simple, b_simple, att_mat, att_vec, Hs_val, Hj_val, Hp_val, R_val, Hs_idx, Hj_idx, Hp_idx, R_idx):
    raise NotImplementedError("write your pallas kernel here")


# speedup vs baseline: 5.3674x; 5.3674x over previous
import functools

import jax
import jax.numpy as jnp
from jax import lax
from jax.experimental import pallas as pl
from jax.experimental.pallas import tpu as pltpu
from jax.experimental.pallas import tpu_sc as plsc

N = 50000
D = 64
HALF = 32
E = 800000
NSUB = 16
NCORE = 2
EPT = E // NSUB
B = 400
NCH = EPT // B
NP = 50048
RPT = NP // NSUB
ZR = 184
GRP = B // 16

_f32 = jnp.float32
_i32 = jnp.int32


def _spmm_sc(table2n, g_idx, s_idx, vals, table_planar):
    mesh = plsc.VectorSubcoreMesh(
        core_axis_name="c", subcore_axis_name="s",
        num_cores=NCORE, num_subcores=NSUB)

    @functools.partial(
        pl.kernel,
        out_type=jax.ShapeDtypeStruct((2 * NP, HALF), _f32),
        mesh=mesh,
        scratch_types=[
            pltpu.VMEM((B,), _i32),
            pltpu.VMEM((B,), _i32),
            pltpu.VMEM((B,), _f32),
            pltpu.VMEM((B, HALF), _f32),
            pltpu.VMEM_SHARED((NP, HALF), _f32),
            pltpu.SemaphoreType.DMA,
        ],
        compiler_params=pltpu.CompilerParams(use_tc_tiling_on_sc=False),
    )
    def body(table_hbm, gi_hbm, si_hbm, val_hbm, out_hbm,
             gi_v, si_v, val_v, gath_v, acc_sh, sem):
        c = lax.axis_index("c")
        s = lax.axis_index("s")
        zeros16 = jnp.zeros((16,), _f32)

        def zero_body(i, _):
            gath_v[i, pl.ds(0, 16)] = zeros16
            gath_v[i, pl.ds(16, 16)] = zeros16
            return 0
        lax.fori_loop(0, ZR, zero_body, 0)
        for k in range(RPT // ZR):
            pltpu.sync_copy(gath_v.at[pl.ds(0, ZR)],
                            acc_sh.at[pl.ds(s * RPT + k * ZR, ZR)])
        plsc.subcore_barrier()

        if table_planar:
            gmul, goff = 1, c * NP
        else:
            gmul, goff = 2, c
        goff_v = jnp.full((16,), goff, _i32)

        def chunk_body(ch, _):
            base = s * EPT + ch * B
            pltpu.sync_copy(gi_hbm.at[pl.ds(base, B)], gi_v)
            pltpu.sync_copy(si_hbm.at[pl.ds(base, B)], si_v)
            pltpu.sync_copy(val_hbm.at[pl.ds(base, B)], val_v)

            def xf_body(g, _):
                t = gi_v[pl.ds(g * 16, 16)]
                gi_v[pl.ds(g * 16, 16)] = t * gmul + goff_v
                return 0
            lax.fori_loop(0, GRP, xf_body, 0)

            pltpu.async_copy(table_hbm.at[gi_v], gath_v, sem).wait()

            def scale_body(g, _):
                e0 = g * 16
                vv = val_v[pl.ds(e0, 16)]
                for j in range(16):
                    bv = jnp.broadcast_to(vv[j], (16,))
                    e = e0 + j
                    gath_v[e, pl.ds(0, 16)] = gath_v[e, pl.ds(0, 16)] * bv
                    gath_v[e, pl.ds(16, 16)] = gath_v[e, pl.ds(16, 16)] * bv
                return 0
            lax.fori_loop(0, GRP, scale_body, 0)

            pltpu.sync_copy(gath_v, acc_sh.at[si_v], add=True)
            return 0
        lax.fori_loop(0, NCH, chunk_body, 0)
        plsc.subcore_barrier()

        for k in range(RPT // ZR):
            r0 = s * RPT + k * ZR
            pltpu.sync_copy(acc_sh.at[pl.ds(r0, ZR)], gath_v.at[pl.ds(0, ZR)])
            pltpu.sync_copy(gath_v.at[pl.ds(0, ZR)],
                            out_hbm.at[pl.ds(c * NP + r0, ZR)])

    return body(table2n, g_idx, s_idx, vals)


RB = 2000
NRB = N // RB


def _attention_mix(u1, u2, u3, att_mat, att_vec):
    a = jnp.dot(att_mat, att_vec.T, preferred_element_type=_f32)
    w1 = jnp.dot(u1, a, preferred_element_type=_f32)
    w2 = jnp.dot(u2, a, preferred_element_type=_f32)
    w3 = jnp.dot(u3, a, preferred_element_type=_f32)
    m = jnp.maximum(jnp.maximum(w1, w2), w3)
    e1 = jnp.exp(w1 - m)
    e2 = jnp.exp(w2 - m)
    e3 = jnp.exp(w3 - m)
    inv = 1.0 / (e1 + e2 + e3)
    return u1 * (e1 * inv) + u2 * (e2 * inv) + u3 * (e3 * inv)


def _gate(x, W, b):
    h = jnp.dot(x, W, preferred_element_type=_f32) + b
    return x * (1.0 / (1.0 + jnp.exp(-h)))


def _row_spec():
    return pl.BlockSpec((RB, D), lambda i: (i, 0))


def _full_spec(shape):
    return pl.BlockSpec(shape, lambda i: tuple(0 for _ in shape))


def _planar_spec(rb=RB):
    return pl.BlockSpec((2, rb, HALF), lambda i: (0, i, 0))


RBC = 1000
NRBC = N // RBC


def _stage_a(user_emb, W1, b1, W2, b2, W3, b3, Ws, bs, att_mat, att_vec):
    def body(x_ref, w1, bb1, w2, bb2, w3, bb3, ws, bbs, am, av,
             o1, o2, o3, os, om):
        x = x_ref[...]
        u1 = _gate(x, w1[...], bb1[...])
        u2 = _gate(x, w2[...], bb2[...])
        u3 = _gate(x, w3[...], bb3[...])
        us = _gate(x, ws[...], bbs[...])
        mixed = _attention_mix(u1, u2, u3, am[...], av[...]) + us * 0.5
        o1[...] = u1
        o2[...] = u2
        o3[...] = u3
        os[...] = us
        om[...] = mixed

    outs = [jax.ShapeDtypeStruct((N, D), _f32)] * 5
    w = _full_spec((D, D))
    b = _full_spec((1, D))
    return pl.pallas_call(
        body,
        grid=(NRB,),
        in_specs=[_row_spec(), w, b, w, b, w, b, w, b, w, _full_spec((1, D))],
        out_specs=[_row_spec()] * 5,
        out_shape=outs,
    )(user_emb, W1, b1.reshape(1, D), W2, b2.reshape(1, D),
      W3, b3.reshape(1, D), Ws, bs.reshape(1, D), att_mat, att_vec)


def _cat(p):
    return jnp.concatenate([p[0], p[1]], axis=-1)


def _stage_b(u1p, u2p, u3p, usp, att_mat, att_vec):
    def body(p1, p2, p3, ps, am, av, om):
        u1 = _cat(p1[...])
        u2 = _cat(p2[...])
        u3 = _cat(p3[...])
        us = _cat(ps[...])
        om[...] = _attention_mix(u1, u2, u3, am[...], av[...]) + us * 0.5

    return pl.pallas_call(
        body,
        grid=(NRB,),
        in_specs=[_planar_spec()] * 4 + [_full_spec((D, D)), _full_spec((1, D))],
        out_specs=_row_spec(),
        out_shape=jax.ShapeDtypeStruct((N, D), _f32),
    )(u1p.reshape(2, NP, HALF), u2p.reshape(2, NP, HALF),
      u3p.reshape(2, NP, HALF), usp.reshape(2, NP, HALF), att_mat, att_vec)


def _normalize(x):
    n = jnp.maximum(jnp.sqrt(jnp.sum(x * x, axis=1, keepdims=True)), 1e-12)
    return x / n


def _stage_c(u10, u20, u30, us0, u11p, u21p, u31p, us1p, i1p,
             u12p, u22p, u32p, us2p, i2p, item_emb, att_mat, att_vec):
    def body(r10, r20, r30, rs0, p11, p21, p31, ps1, pi1,
             p12, p22, p32, ps2, pi2, rie, am, av, ou, oi):
        u1f = r10[...] + _normalize(_cat(p11[...])) + _normalize(_cat(p12[...]))
        u2f = r20[...] + _normalize(_cat(p21[...])) + _normalize(_cat(p22[...]))
        u3f = r30[...] + _normalize(_cat(p31[...])) + _normalize(_cat(p32[...]))
        usf = rs0[...] + _normalize(_cat(ps1[...])) + _normalize(_cat(ps2[...]))
        ou[...] = _attention_mix(u1f, u2f, u3f, am[...], av[...]) + usf * 0.5
        oi[...] = (rie[...] + _normalize(_cat(pi1[...]))
                   + _normalize(_cat(pi2[...])))

    row_c = pl.BlockSpec((RBC, D), lambda i: (i, 0))
    return pl.pallas_call(
        body,
        grid=(NRBC,),
        in_specs=([row_c] * 4 + [_planar_spec(RBC)] * 10
                  + [row_c, _full_spec((D, D)), _full_spec((1, D))]),
        out_specs=[row_c] * 2,
        out_shape=[jax.ShapeDtypeStruct((N, D), _f32)] * 2,
    )(u10, u20, u30, us0,
      u11p.reshape(2, NP, HALF), u21p.reshape(2, NP, HALF),
      u31p.reshape(2, NP, HALF), us1p.reshape(2, NP, HALF),
      i1p.reshape(2, NP, HALF),
      u12p.reshape(2, NP, HALF), u22p.reshape(2, NP, HALF),
      u32p.reshape(2, NP, HALF), us2p.reshape(2, NP, HALF),
      i2p.reshape(2, NP, HALF),
      item_emb, att_mat, att_vec)


def kernel(user_emb, item_emb, W_c1, b_c1, W_c2, b_c2, W_c3, b_c3,
           W_simple, b_simple, att_mat, att_vec,
           Hs_val, Hj_val, Hp_val, R_val,
           Hs_idx, Hj_idx, Hp_idx, R_idx):
    u10, u20, u30, us0, m1 = _stage_a(
        user_emb, W_c1, b_c1, W_c2, b_c2, W_c3, b_c3, W_simple, b_simple,
        att_mat, att_vec)

    hs_r, hs_c = Hs_idx[0], Hs_idx[1]
    hj_r, hj_c = Hj_idx[0], Hj_idx[1]
    hp_r, hp_c = Hp_idx[0], Hp_idx[1]
    r_r, r_c = R_idx[0], R_idx[1]

    def v(x):
        return x.reshape(2 * N, HALF)

    u11p = _spmm_sc(v(u10), hs_c, hs_r, Hs_val, table_planar=False)
    u21p = _spmm_sc(v(u20), hj_c, hj_r, Hj_val, table_planar=False)
    u31p = _spmm_sc(v(u30), hp_c, hp_r, Hp_val, table_planar=False)
    i1p = _spmm_sc(v(m1), r_r, r_c, R_val, table_planar=False)
    us1p = _spmm_sc(v(item_emb), r_c, r_r, R_val, table_planar=False)

    m2 = _stage_b(u11p, u21p, u31p, us1p, att_mat, att_vec)

    u12p = _spmm_sc(u11p, hs_c, hs_r, Hs_val, table_planar=True)
    u22p = _spmm_sc(u21p, hj_c, hj_r, Hj_val, table_planar=True)
    u32p = _spmm_sc(u31p, hp_c, hp_r, Hp_val, table_planar=True)
    i2p = _spmm_sc(v(m2), r_r, r_c, R_val, table_planar=False)
    us2p = _spmm_sc(i1p, r_c, r_r, R_val, table_planar=True)

    user_all, item_all = _stage_c(
        u10, u20, u30, us0, u11p, u21p, u31p, us1p, i1p,
        u12p, u22p, u32p, us2p, i2p, item_emb, att_mat, att_vec)
    return (user_all, item_all)


# speedup vs baseline: 12.1645x; 2.2664x over previous
import functools

import jax
import jax.numpy as jnp
from jax import lax
from jax.experimental import pallas as pl
from jax.experimental.pallas import tpu as pltpu
from jax.experimental.pallas import tpu_sc as plsc

N = 50000
D = 64
HALF = 32
E = 800000
NSUB = 16
NCORE = 2
EPT = E // NSUB
B = 400
NCH = EPT // B
NP = 50048
RPT = NP // NSUB
ZR = 184
GRP = B // 16

_f32 = jnp.float32
_i32 = jnp.int32


def _spmm_sc(table2n, g_idx, s_idx, vals, table_planar):
    mesh = plsc.VectorSubcoreMesh(
        core_axis_name="c", subcore_axis_name="s",
        num_cores=NCORE, num_subcores=NSUB)

    @functools.partial(
        pl.kernel,
        out_type=jax.ShapeDtypeStruct((2 * NP, HALF), _f32),
        mesh=mesh,
        scratch_types=[
            pltpu.VMEM((B,), _i32), pltpu.VMEM((B,), _i32),
            pltpu.VMEM((B,), _i32), pltpu.VMEM((B,), _i32),
            pltpu.VMEM((B,), _f32), pltpu.VMEM((B,), _f32),
            pltpu.VMEM((B, HALF), _f32), pltpu.VMEM((B, HALF), _f32),
            pltpu.VMEM_SHARED((NP, HALF), _f32),
            pltpu.SemaphoreType.DMA, pltpu.SemaphoreType.DMA,
            pltpu.SemaphoreType.DMA, pltpu.SemaphoreType.DMA,
            pltpu.SemaphoreType.DMA, pltpu.SemaphoreType.DMA,
            pltpu.SemaphoreType.DMA, pltpu.SemaphoreType.DMA,
        ],
        compiler_params=pltpu.CompilerParams(use_tc_tiling_on_sc=False),
    )
    def body(table_hbm, gi_hbm, si_hbm, val_hbm, out_hbm,
             gi0, gi1, si0, si1, val0, val1, gath0, gath1, acc_sh,
             sem_iv0, sem_iv1, sem_si0, sem_si1,
             sem_g0, sem_g1, sem_s0, sem_s1):
        gi = (gi0, gi1)
        si = (si0, si1)
        val = (val0, val1)
        gath = (gath0, gath1)
        sem_iv = (sem_iv0, sem_iv1)
        sem_si = (sem_si0, sem_si1)
        sem_g = (sem_g0, sem_g1)
        sem_s = (sem_s0, sem_s1)
        c = lax.axis_index("c")
        s = lax.axis_index("s")
        zeros16 = jnp.zeros((16,), _f32)

        def zero_body(i, _):
            gath0[i, pl.ds(0, 16)] = zeros16
            gath0[i, pl.ds(16, 16)] = zeros16
            return 0
        lax.fori_loop(0, ZR, zero_body, 0)
        for k in range(RPT // ZR):
            pltpu.sync_copy(gath0.at[pl.ds(0, ZR)],
                            acc_sh.at[pl.ds(s * RPT + k * ZR, ZR)])
        plsc.subcore_barrier()

        if table_planar:
            gmul, goff = 1, c * NP
        else:
            gmul, goff = 2, c
        goff_v = jnp.full((16,), goff, _i32)
        ebase = s * EPT

        def gv_start(ch, p):
            pltpu.async_copy(gi_hbm.at[pl.ds(ebase + ch * B, B)], gi[p],
                             sem_iv[p])
            pltpu.async_copy(val_hbm.at[pl.ds(ebase + ch * B, B)], val[p],
                             sem_iv[p])

        def gv_wait(ch, p):
            pltpu.make_async_copy(gi_hbm.at[pl.ds(ebase + ch * B, B)], gi[p],
                                  sem_iv[p]).wait()
            pltpu.make_async_copy(val_hbm.at[pl.ds(ebase + ch * B, B)], val[p],
                                  sem_iv[p]).wait()

        def si_start(ch, p):
            pltpu.async_copy(si_hbm.at[pl.ds(ebase + ch * B, B)], si[p],
                             sem_si[p])

        def si_wait(ch, p):
            pltpu.make_async_copy(si_hbm.at[pl.ds(ebase + ch * B, B)], si[p],
                                  sem_si[p]).wait()

        def transform(p):
            def xf_body(g, _):
                t = gi[p][pl.ds(g * 16, 16)]
                gi[p][pl.ds(g * 16, 16)] = t * gmul + goff_v
                return 0
            lax.fori_loop(0, GRP, xf_body, 0)

        def g_start(p):
            pltpu.async_copy(table_hbm.at[gi[p]], gath[p], sem_g[p])

        def g_wait(p):
            pltpu.make_async_copy(table_hbm.at[gi[p]], gath[p],
                                  sem_g[p]).wait()

        def scale(p):
            gv = gath[p]
            vr = val[p]

            def scale_body(g, _):
                e0 = g * 16
                vv = vr[pl.ds(e0, 16)]
                for j in range(16):
                    bv = jnp.broadcast_to(vv[j], (16,))
                    e = e0 + j
                    gv[e, pl.ds(0, 16)] = gv[e, pl.ds(0, 16)] * bv
                    gv[e, pl.ds(16, 16)] = gv[e, pl.ds(16, 16)] * bv
                return 0
            lax.fori_loop(0, GRP, scale_body, 0)

        def s_start(p):
            pltpu.async_copy(gath[p], acc_sh.at[si[p]], sem_s[p], add=True)

        def s_wait(p):
            pltpu.make_async_copy(gath[p], acc_sh.at[si[p]], sem_s[p]).wait()

        def process(ch, p, k, first=False, prefetch_pred=None):
            q = 1 - p
            g_wait(p)
            if first:
                si_start(ch + 1, q)
            else:
                def do_sw():
                    s_wait(q)
                    si_start(ch + 1, q)
                if k is None:
                    do_sw()
                else:
                    pl.when(k > 0)(do_sw)
                    pl.when(k == 0)(lambda: si_start(ch + 1, q))
            gv_wait(ch + 1, q)
            transform(q)
            g_start(q)
            scale(p)
            if prefetch_pred is None:
                gv_start(ch + 2, p)
            else:
                pl.when(prefetch_pred)(lambda: gv_start(ch + 2, p))
            si_wait(ch, p)
            s_start(p)

        gv_start(0, 0)
        si_start(0, 0)
        gv_start(1, 1)
        gv_wait(0, 0)
        transform(0)
        g_start(0)

        def pair_body(k, _):
            a = 2 * k
            process(a, 0, k)
            process(a + 1, 1, None, prefetch_pred=(k < NCH // 2 - 1))
            return 0
        lax.fori_loop(0, NCH // 2, pair_body, 0)

        g_wait(0)
        s_wait(1)
        scale(0)
        si_wait(NCH - 1, 0)
        s_start(0)
        s_wait(0)
        plsc.subcore_barrier()

        for k in range(RPT // ZR):
            r0 = s * RPT + k * ZR
            pltpu.sync_copy(acc_sh.at[pl.ds(r0, ZR)], gath0.at[pl.ds(0, ZR)])
            pltpu.sync_copy(gath0.at[pl.ds(0, ZR)],
                            out_hbm.at[pl.ds(c * NP + r0, ZR)])

    return body(table2n, g_idx, s_idx, vals)


RB = 2000
NRB = N // RB


def _attention_mix(u1, u2, u3, att_mat, att_vec):
    a = jnp.dot(att_mat, att_vec.T, preferred_element_type=_f32)
    w1 = jnp.dot(u1, a, preferred_element_type=_f32)
    w2 = jnp.dot(u2, a, preferred_element_type=_f32)
    w3 = jnp.dot(u3, a, preferred_element_type=_f32)
    m = jnp.maximum(jnp.maximum(w1, w2), w3)
    e1 = jnp.exp(w1 - m)
    e2 = jnp.exp(w2 - m)
    e3 = jnp.exp(w3 - m)
    inv = 1.0 / (e1 + e2 + e3)
    return u1 * (e1 * inv) + u2 * (e2 * inv) + u3 * (e3 * inv)


def _gate(x, W, b):
    h = jnp.dot(x, W, preferred_element_type=_f32) + b
    return x * (1.0 / (1.0 + jnp.exp(-h)))


def _row_spec():
    return pl.BlockSpec((RB, D), lambda i: (i, 0))


def _full_spec(shape):
    return pl.BlockSpec(shape, lambda i: tuple(0 for _ in shape))


def _planar_spec(rb=RB):
    return pl.BlockSpec((2, rb, HALF), lambda i: (0, i, 0))


RBC = 1000
NRBC = N // RBC


def _stage_a(user_emb, W1, b1, W2, b2, W3, b3, Ws, bs, att_mat, att_vec):
    def body(x_ref, w1, bb1, w2, bb2, w3, bb3, ws, bbs, am, av,
             o1, o2, o3, os, om):
        x = x_ref[...]
        u1 = _gate(x, w1[...], bb1[...])
        u2 = _gate(x, w2[...], bb2[...])
        u3 = _gate(x, w3[...], bb3[...])
        us = _gate(x, ws[...], bbs[...])
        mixed = _attention_mix(u1, u2, u3, am[...], av[...]) + us * 0.5
        o1[...] = u1
        o2[...] = u2
        o3[...] = u3
        os[...] = us
        om[...] = mixed

    outs = [jax.ShapeDtypeStruct((N, D), _f32)] * 5
    w = _full_spec((D, D))
    b = _full_spec((1, D))
    return pl.pallas_call(
        body,
        grid=(NRB,),
        in_specs=[_row_spec(), w, b, w, b, w, b, w, b, w, _full_spec((1, D))],
        out_specs=[_row_spec()] * 5,
        out_shape=outs,
    )(user_emb, W1, b1.reshape(1, D), W2, b2.reshape(1, D),
      W3, b3.reshape(1, D), Ws, bs.reshape(1, D), att_mat, att_vec)


def _cat(p):
    return jnp.concatenate([p[0], p[1]], axis=-1)


def _stage_b(u1p, u2p, u3p, usp, att_mat, att_vec):
    def body(p1, p2, p3, ps, am, av, om):
        u1 = _cat(p1[...])
        u2 = _cat(p2[...])
        u3 = _cat(p3[...])
        us = _cat(ps[...])
        om[...] = _attention_mix(u1, u2, u3, am[...], av[...]) + us * 0.5

    return pl.pallas_call(
        body,
        grid=(NRB,),
        in_specs=[_planar_spec()] * 4 + [_full_spec((D, D)), _full_spec((1, D))],
        out_specs=_row_spec(),
        out_shape=jax.ShapeDtypeStruct((N, D), _f32),
    )(u1p.reshape(2, NP, HALF), u2p.reshape(2, NP, HALF),
      u3p.reshape(2, NP, HALF), usp.reshape(2, NP, HALF), att_mat, att_vec)


def _normalize(x):
    n = jnp.maximum(jnp.sqrt(jnp.sum(x * x, axis=1, keepdims=True)), 1e-12)
    return x / n


def _stage_c(u10, u20, u30, us0, u11p, u21p, u31p, us1p, i1p,
             u12p, u22p, u32p, us2p, i2p, item_emb, att_mat, att_vec):
    def body(r10, r20, r30, rs0, p11, p21, p31, ps1, pi1,
             p12, p22, p32, ps2, pi2, rie, am, av, ou, oi):
        u1f = r10[...] + _normalize(_cat(p11[...])) + _normalize(_cat(p12[...]))
        u2f = r20[...] + _normalize(_cat(p21[...])) + _normalize(_cat(p22[...]))
        u3f = r30[...] + _normalize(_cat(p31[...])) + _normalize(_cat(p32[...]))
        usf = rs0[...] + _normalize(_cat(ps1[...])) + _normalize(_cat(ps2[...]))
        ou[...] = _attention_mix(u1f, u2f, u3f, am[...], av[...]) + usf * 0.5
        oi[...] = (rie[...] + _normalize(_cat(pi1[...]))
                   + _normalize(_cat(pi2[...])))

    row_c = pl.BlockSpec((RBC, D), lambda i: (i, 0))
    return pl.pallas_call(
        body,
        grid=(NRBC,),
        in_specs=([row_c] * 4 + [_planar_spec(RBC)] * 10
                  + [row_c, _full_spec((D, D)), _full_spec((1, D))]),
        out_specs=[row_c] * 2,
        out_shape=[jax.ShapeDtypeStruct((N, D), _f32)] * 2,
    )(u10, u20, u30, us0,
      u11p.reshape(2, NP, HALF), u21p.reshape(2, NP, HALF),
      u31p.reshape(2, NP, HALF), us1p.reshape(2, NP, HALF),
      i1p.reshape(2, NP, HALF),
      u12p.reshape(2, NP, HALF), u22p.reshape(2, NP, HALF),
      u32p.reshape(2, NP, HALF), us2p.reshape(2, NP, HALF),
      i2p.reshape(2, NP, HALF),
      item_emb, att_mat, att_vec)


def kernel(user_emb, item_emb, W_c1, b_c1, W_c2, b_c2, W_c3, b_c3,
           W_simple, b_simple, att_mat, att_vec,
           Hs_val, Hj_val, Hp_val, R_val,
           Hs_idx, Hj_idx, Hp_idx, R_idx):
    u10, u20, u30, us0, m1 = _stage_a(
        user_emb, W_c1, b_c1, W_c2, b_c2, W_c3, b_c3, W_simple, b_simple,
        att_mat, att_vec)

    hs_r, hs_c = Hs_idx[0], Hs_idx[1]
    hj_r, hj_c = Hj_idx[0], Hj_idx[1]
    hp_r, hp_c = Hp_idx[0], Hp_idx[1]
    r_r, r_c = R_idx[0], R_idx[1]

    def v(x):
        return x.reshape(2 * N, HALF)

    u11p = _spmm_sc(v(u10), hs_c, hs_r, Hs_val, table_planar=False)
    u21p = _spmm_sc(v(u20), hj_c, hj_r, Hj_val, table_planar=False)
    u31p = _spmm_sc(v(u30), hp_c, hp_r, Hp_val, table_planar=False)
    i1p = _spmm_sc(v(m1), r_r, r_c, R_val, table_planar=False)
    us1p = _spmm_sc(v(item_emb), r_c, r_r, R_val, table_planar=False)

    m2 = _stage_b(u11p, u21p, u31p, us1p, att_mat, att_vec)

    u12p = _spmm_sc(u11p, hs_c, hs_r, Hs_val, table_planar=True)
    u22p = _spmm_sc(u21p, hj_c, hj_r, Hj_val, table_planar=True)
    u32p = _spmm_sc(u31p, hp_c, hp_r, Hp_val, table_planar=True)
    i2p = _spmm_sc(v(m2), r_r, r_c, R_val, table_planar=False)
    us2p = _spmm_sc(i1p, r_c, r_r, R_val, table_planar=True)

    user_all, item_all = _stage_c(
        u10, u20, u30, us0, u11p, u21p, u31p, us1p, i1p,
        u12p, u22p, u32p, us2p, i2p, item_emb, att_mat, att_vec)
    return (user_all, item_all)


# speedup vs baseline: 12.4986x; 1.0275x over previous
import functools

import jax
import jax.numpy as jnp
from jax import lax
from jax.experimental import pallas as pl
from jax.experimental.pallas import tpu as pltpu
from jax.experimental.pallas import tpu_sc as plsc

N = 50000
D = 64
HALF = 32
E = 800000
NSUB = 16
NCORE = 2
EPT = E // NSUB
B = 400
NCH = EPT // B
NP = 50048
RPT = NP // NSUB
ZR = 184
GRP = B // 16

_f32 = jnp.float32
_i32 = jnp.int32


def _spmm_sc(table2n, g_idx, s_idx, vals, table_planar):
    mesh = plsc.VectorSubcoreMesh(
        core_axis_name="c", subcore_axis_name="s",
        num_cores=NCORE, num_subcores=NSUB)

    @functools.partial(
        pl.kernel,
        out_type=jax.ShapeDtypeStruct((2 * NP, HALF), _f32),
        mesh=mesh,
        scratch_types=[
            pltpu.VMEM((B,), _i32), pltpu.VMEM((B,), _i32),
            pltpu.VMEM((B,), _i32), pltpu.VMEM((B,), _i32),
            pltpu.VMEM((B,), _f32), pltpu.VMEM((B,), _f32),
            pltpu.VMEM((B, HALF), _f32), pltpu.VMEM((B, HALF), _f32),
            pltpu.VMEM_SHARED((NP, HALF), _f32),
            pltpu.SemaphoreType.DMA, pltpu.SemaphoreType.DMA,
            pltpu.SemaphoreType.DMA, pltpu.SemaphoreType.DMA,
            pltpu.SemaphoreType.DMA, pltpu.SemaphoreType.DMA,
            pltpu.SemaphoreType.DMA, pltpu.SemaphoreType.DMA,
        ],
        compiler_params=pltpu.CompilerParams(use_tc_tiling_on_sc=False),
    )
    def body(table_hbm, gi_hbm, si_hbm, val_hbm, out_hbm,
             gi0, gi1, si0, si1, val0, val1, gath0, gath1, acc_sh,
             sem_iv0, sem_iv1, sem_si0, sem_si1,
             sem_g0, sem_g1, sem_s0, sem_s1):
        gi = (gi0, gi1)
        si = (si0, si1)
        val = (val0, val1)
        gath = (gath0, gath1)
        sem_iv = (sem_iv0, sem_iv1)
        sem_si = (sem_si0, sem_si1)
        sem_g = (sem_g0, sem_g1)
        sem_s = (sem_s0, sem_s1)
        c = lax.axis_index("c")
        s = lax.axis_index("s")
        zeros16 = jnp.zeros((16,), _f32)

        def zero_body(i, _):
            gath0[i, pl.ds(0, 16)] = zeros16
            gath0[i, pl.ds(16, 16)] = zeros16
            return 0
        lax.fori_loop(0, ZR, zero_body, 0)
        for k in range(RPT // ZR):
            pltpu.sync_copy(gath0.at[pl.ds(0, ZR)],
                            acc_sh.at[pl.ds(s * RPT + k * ZR, ZR)])
        plsc.subcore_barrier()

        if table_planar:
            gmul, goff = 1, c * NP
        else:
            gmul, goff = 2, c
        goff_v = jnp.full((16,), goff, _i32)
        ebase = s * EPT

        def gv_start(ch, p):
            pltpu.async_copy(gi_hbm.at[pl.ds(ebase + ch * B, B)], gi[p],
                             sem_iv[p])
            pltpu.async_copy(val_hbm.at[pl.ds(ebase + ch * B, B)], val[p],
                             sem_iv[p])

        def gv_wait(ch, p):
            pltpu.make_async_copy(gi_hbm.at[pl.ds(ebase + ch * B, B)], gi[p],
                                  sem_iv[p]).wait()
            pltpu.make_async_copy(val_hbm.at[pl.ds(ebase + ch * B, B)], val[p],
                                  sem_iv[p]).wait()

        def si_start(ch, p):
            pltpu.async_copy(si_hbm.at[pl.ds(ebase + ch * B, B)], si[p],
                             sem_si[p])

        def si_wait(ch, p):
            pltpu.make_async_copy(si_hbm.at[pl.ds(ebase + ch * B, B)], si[p],
                                  sem_si[p]).wait()

        def transform(p):
            @plsc.parallel_loop(0, GRP, unroll=2)
            def _(g):
                t = gi[p][pl.ds(g * 16, 16)]
                gi[p][pl.ds(g * 16, 16)] = t * gmul + goff_v

        def g_start(p):
            pltpu.async_copy(table_hbm.at[gi[p]], gath[p], sem_g[p])

        def g_wait(p):
            pltpu.make_async_copy(table_hbm.at[gi[p]], gath[p],
                                  sem_g[p]).wait()

        def scale(p):
            gv = gath[p]
            vr = val[p]

            @plsc.parallel_loop(0, GRP, unroll=2)
            def _(g):
                e0 = g * 16
                vv = vr[pl.ds(e0, 16)]
                for j in range(16):
                    bv = jnp.broadcast_to(vv[j], (16,))
                    e = e0 + j
                    gv[e, pl.ds(0, 16)] = gv[e, pl.ds(0, 16)] * bv
                    gv[e, pl.ds(16, 16)] = gv[e, pl.ds(16, 16)] * bv

        def s_start(p):
            pltpu.async_copy(gath[p], acc_sh.at[si[p]], sem_s[p], add=True)

        def s_wait(p):
            pltpu.make_async_copy(gath[p], acc_sh.at[si[p]], sem_s[p]).wait()

        def process(ch, p, k, first=False, prefetch_pred=None):
            q = 1 - p
            g_wait(p)
            if first:
                si_start(ch + 1, q)
            else:
                def do_sw():
                    s_wait(q)
                    si_start(ch + 1, q)
                if k is None:
                    do_sw()
                else:
                    pl.when(k > 0)(do_sw)
                    pl.when(k == 0)(lambda: si_start(ch + 1, q))
            gv_wait(ch + 1, q)
            transform(q)
            g_start(q)
            scale(p)
            if prefetch_pred is None:
                gv_start(ch + 2, p)
            else:
                pl.when(prefetch_pred)(lambda: gv_start(ch + 2, p))
            si_wait(ch, p)
            s_start(p)

        gv_start(0, 0)
        si_start(0, 0)
        gv_start(1, 1)
        gv_wait(0, 0)
        transform(0)
        g_start(0)

        def pair_body(k, _):
            a = 2 * k
            process(a, 0, k)
            process(a + 1, 1, None, prefetch_pred=(k < NCH // 2 - 1))
            return 0
        lax.fori_loop(0, NCH // 2, pair_body, 0)

        g_wait(0)
        s_wait(1)
        scale(0)
        si_wait(NCH - 1, 0)
        s_start(0)
        s_wait(0)
        plsc.subcore_barrier()

        for k in range(RPT // ZR):
            r0 = s * RPT + k * ZR
            pltpu.sync_copy(acc_sh.at[pl.ds(r0, ZR)], gath0.at[pl.ds(0, ZR)])
            pltpu.sync_copy(gath0.at[pl.ds(0, ZR)],
                            out_hbm.at[pl.ds(c * NP + r0, ZR)])

    return body(table2n, g_idx, s_idx, vals)


RB = 2000
NRB = N // RB


def _attention_mix(u1, u2, u3, att_mat, att_vec):
    a = jnp.dot(att_mat, att_vec.T, preferred_element_type=_f32)
    w1 = jnp.dot(u1, a, preferred_element_type=_f32)
    w2 = jnp.dot(u2, a, preferred_element_type=_f32)
    w3 = jnp.dot(u3, a, preferred_element_type=_f32)
    m = jnp.maximum(jnp.maximum(w1, w2), w3)
    e1 = jnp.exp(w1 - m)
    e2 = jnp.exp(w2 - m)
    e3 = jnp.exp(w3 - m)
    inv = 1.0 / (e1 + e2 + e3)
    return u1 * (e1 * inv) + u2 * (e2 * inv) + u3 * (e3 * inv)


def _gate(x, W, b):
    h = jnp.dot(x, W, preferred_element_type=_f32) + b
    return x * (1.0 / (1.0 + jnp.exp(-h)))


def _row_spec():
    return pl.BlockSpec((RB, D), lambda i: (i, 0))


def _full_spec(shape):
    return pl.BlockSpec(shape, lambda i: tuple(0 for _ in shape))


def _planar_spec(rb=RB):
    return pl.BlockSpec((2, rb, HALF), lambda i: (0, i, 0))


RBC = 1000
NRBC = N // RBC


def _stage_a(user_emb, W1, b1, W2, b2, W3, b3, Ws, bs, att_mat, att_vec):
    def body(x_ref, w1, bb1, w2, bb2, w3, bb3, ws, bbs, am, av,
             o1, o2, o3, os, om):
        x = x_ref[...]
        u1 = _gate(x, w1[...], bb1[...])
        u2 = _gate(x, w2[...], bb2[...])
        u3 = _gate(x, w3[...], bb3[...])
        us = _gate(x, ws[...], bbs[...])
        mixed = _attention_mix(u1, u2, u3, am[...], av[...]) + us * 0.5
        o1[...] = u1
        o2[...] = u2
        o3[...] = u3
        os[...] = us
        om[...] = mixed

    outs = [jax.ShapeDtypeStruct((N, D), _f32)] * 5
    w = _full_spec((D, D))
    b = _full_spec((1, D))
    return pl.pallas_call(
        body,
        grid=(NRB,),
        in_specs=[_row_spec(), w, b, w, b, w, b, w, b, w, _full_spec((1, D))],
        out_specs=[_row_spec()] * 5,
        out_shape=outs,
    )(user_emb, W1, b1.reshape(1, D), W2, b2.reshape(1, D),
      W3, b3.reshape(1, D), Ws, bs.reshape(1, D), att_mat, att_vec)


def _cat(p):
    return jnp.concatenate([p[0], p[1]], axis=-1)


def _stage_b(u1p, u2p, u3p, usp, att_mat, att_vec):
    def body(p1, p2, p3, ps, am, av, om):
        u1 = _cat(p1[...])
        u2 = _cat(p2[...])
        u3 = _cat(p3[...])
        us = _cat(ps[...])
        om[...] = _attention_mix(u1, u2, u3, am[...], av[...]) + us * 0.5

    return pl.pallas_call(
        body,
        grid=(NRB,),
        in_specs=[_planar_spec()] * 4 + [_full_spec((D, D)), _full_spec((1, D))],
        out_specs=_row_spec(),
        out_shape=jax.ShapeDtypeStruct((N, D), _f32),
    )(u1p.reshape(2, NP, HALF), u2p.reshape(2, NP, HALF),
      u3p.reshape(2, NP, HALF), usp.reshape(2, NP, HALF), att_mat, att_vec)


def _normalize(x):
    n = jnp.maximum(jnp.sqrt(jnp.sum(x * x, axis=1, keepdims=True)), 1e-12)
    return x / n


def _stage_c(u10, u20, u30, us0, u11p, u21p, u31p, us1p, i1p,
             u12p, u22p, u32p, us2p, i2p, item_emb, att_mat, att_vec):
    def body(r10, r20, r30, rs0, p11, p21, p31, ps1, pi1,
             p12, p22, p32, ps2, pi2, rie, am, av, ou, oi):
        u1f = r10[...] + _normalize(_cat(p11[...])) + _normalize(_cat(p12[...]))
        u2f = r20[...] + _normalize(_cat(p21[...])) + _normalize(_cat(p22[...]))
        u3f = r30[...] + _normalize(_cat(p31[...])) + _normalize(_cat(p32[...]))
        usf = rs0[...] + _normalize(_cat(ps1[...])) + _normalize(_cat(ps2[...]))
        ou[...] = _attention_mix(u1f, u2f, u3f, am[...], av[...]) + usf * 0.5
        oi[...] = (rie[...] + _normalize(_cat(pi1[...]))
                   + _normalize(_cat(pi2[...])))

    row_c = pl.BlockSpec((RBC, D), lambda i: (i, 0))
    return pl.pallas_call(
        body,
        grid=(NRBC,),
        in_specs=([row_c] * 4 + [_planar_spec(RBC)] * 10
                  + [row_c, _full_spec((D, D)), _full_spec((1, D))]),
        out_specs=[row_c] * 2,
        out_shape=[jax.ShapeDtypeStruct((N, D), _f32)] * 2,
    )(u10, u20, u30, us0,
      u11p.reshape(2, NP, HALF), u21p.reshape(2, NP, HALF),
      u31p.reshape(2, NP, HALF), us1p.reshape(2, NP, HALF),
      i1p.reshape(2, NP, HALF),
      u12p.reshape(2, NP, HALF), u22p.reshape(2, NP, HALF),
      u32p.reshape(2, NP, HALF), us2p.reshape(2, NP, HALF),
      i2p.reshape(2, NP, HALF),
      item_emb, att_mat, att_vec)


def kernel(user_emb, item_emb, W_c1, b_c1, W_c2, b_c2, W_c3, b_c3,
           W_simple, b_simple, att_mat, att_vec,
           Hs_val, Hj_val, Hp_val, R_val,
           Hs_idx, Hj_idx, Hp_idx, R_idx):
    u10, u20, u30, us0, m1 = _stage_a(
        user_emb, W_c1, b_c1, W_c2, b_c2, W_c3, b_c3, W_simple, b_simple,
        att_mat, att_vec)

    hs_r, hs_c = Hs_idx[0], Hs_idx[1]
    hj_r, hj_c = Hj_idx[0], Hj_idx[1]
    hp_r, hp_c = Hp_idx[0], Hp_idx[1]
    r_r, r_c = R_idx[0], R_idx[1]

    def v(x):
        return x.reshape(2 * N, HALF)

    u11p = _spmm_sc(v(u10), hs_c, hs_r, Hs_val, table_planar=False)
    u21p = _spmm_sc(v(u20), hj_c, hj_r, Hj_val, table_planar=False)
    u31p = _spmm_sc(v(u30), hp_c, hp_r, Hp_val, table_planar=False)
    i1p = _spmm_sc(v(m1), r_r, r_c, R_val, table_planar=False)
    us1p = _spmm_sc(v(item_emb), r_c, r_r, R_val, table_planar=False)

    m2 = _stage_b(u11p, u21p, u31p, us1p, att_mat, att_vec)

    u12p = _spmm_sc(u11p, hs_c, hs_r, Hs_val, table_planar=True)
    u22p = _spmm_sc(u21p, hj_c, hj_r, Hj_val, table_planar=True)
    u32p = _spmm_sc(u31p, hp_c, hp_r, Hp_val, table_planar=True)
    i2p = _spmm_sc(v(m2), r_r, r_c, R_val, table_planar=False)
    us2p = _spmm_sc(i1p, r_c, r_r, R_val, table_planar=True)

    user_all, item_all = _stage_c(
        u10, u20, u30, us0, u11p, u21p, u31p, us1p, i1p,
        u12p, u22p, u32p, us2p, i2p, item_emb, att_mat, att_vec)
    return (user_all, item_all)
